# Initial kernel scaffold; baseline (speedup 1.0000x reference)
import jax
import jax.numpy as jnp
from jax.experimental import pallas as pl


def kernel(X, cluster_assignment):
    raise NotImplementedError("write your pallas kernel here")


# speedup vs baseline: 2.4055x; 2.4055x over previous
import jax
import jax.numpy as jnp
from jax import lax
from jax.experimental import pallas as pl
from jax.experimental.pallas import tpu as pltpu
from jax.experimental.pallas import tpu_sc as plsc

N = 320000
D = 128
S = 10000
L = 16
NW = 32
SEG_PER = 313
S_PAD = NW * SEG_PER
C = 256
NB = 48


def _sc_body(x_hbm, bounds_hbm, ids_hbm, out_hbm,
             acc_ref, xbuf_ref, ids_vmem, bounds_vmem, counts_smem):
    w = lax.axis_index("c") * 16 + lax.axis_index("s")
    s_lo = w * SEG_PER

    pltpu.sync_copy(bounds_hbm, bounds_vmem.at[pl.ds(0, NB)])
    bv = bounds_vmem[pl.ds(w, L)]
    r_lo = bv[0]
    r_hi = bv[1]

    @pl.loop(0, SEG_PER * D, step=L)
    def _(o):
        acc_ref[pl.ds(o, L)] = jnp.zeros((L,), jnp.float32)

    @pl.loop(0, SEG_PER)
    def _(k):
        counts_smem[k] = 0

    base = (r_lo // 16) * 16
    nchunks = (r_hi - base + C - 1) // C

    def chunk_body(cix, _):
        start = jnp.minimum(base + cix * C, N - C)
        pltpu.sync_copy(x_hbm.at[pl.ds(start * D, C * D)], xbuf_ref)
        pltpu.sync_copy(ids_hbm.at[pl.ds(start, C)], ids_vmem.at[pl.ds(0, C)])
        lo = jnp.maximum(r_lo, base + cix * C)
        hi = jnp.minimum(base + (cix + 1) * C, r_hi)

        def row_body(r, _):
            i = r - start
            lid = ids_vmem[pl.ds(i, L)][0] - s_lo
            counts_smem[lid] = counts_smem[lid] + 1
            off = lid * D
            ioff = i * D
            for j in range(D // L):
                v = xbuf_ref[pl.ds(ioff + j * L, L)]
                plsc.addupdate(acc_ref.at[pl.ds(off + j * L, L)], v)
            return 0

        lax.fori_loop(lo, hi, row_body, 0)
        return 0

    lax.fori_loop(0, nchunks, chunk_body, 0)

    @pl.loop(0, SEG_PER)
    def _(k):
        cntf = jnp.maximum(counts_smem[k], 1).astype(jnp.float32)
        off = k * D
        for j in range(D // L):
            v = acc_ref[pl.ds(off + j * L, L)]
            acc_ref[pl.ds(off + j * L, L)] = v / cntf

    pltpu.sync_copy(acc_ref, out_hbm.at[pl.ds(s_lo * D, SEG_PER * D)])


def _make_sc_kernel(interpret=False):
    mesh = plsc.VectorSubcoreMesh(
        core_axis_name="c", subcore_axis_name="s", num_cores=2, num_subcores=16
    )
    return pl.kernel(
        _sc_body,
        out_type=jax.ShapeDtypeStruct((S_PAD * D,), jnp.float32),
        mesh=mesh,
        scratch_types=[
            pltpu.VMEM((SEG_PER * D,), jnp.float32),
            pltpu.VMEM((C * D,), jnp.float32),
            pltpu.VMEM((C + L,), jnp.int32),
            pltpu.VMEM((NB + L,), jnp.int32),
            pltpu.SMEM((SEG_PER,), jnp.int32),
        ],
        interpret=interpret,
    )


@jax.jit
def kernel(X, cluster_assignment):
    ids = cluster_assignment.astype(jnp.int32)
    queries = jnp.arange(NB, dtype=jnp.int32) * SEG_PER
    bounds = jnp.searchsorted(ids, queries, side="left").astype(jnp.int32)
    out = _make_sc_kernel()(X.reshape(N * D), bounds, ids)
    return out.reshape(S_PAD, D)[:S]


# speedup vs baseline: 2.8858x; 1.1997x over previous
import jax
import jax.numpy as jnp
from jax import lax
from jax.experimental import pallas as pl
from jax.experimental.pallas import tpu as pltpu
from jax.experimental.pallas import tpu_sc as plsc

N = 320000
D = 128
S = 10000
L = 16
NW = 32
SEG_PER = 313
S_PAD = NW * SEG_PER
C = 320
NB = 48


def _sc_body(x_hbm, bounds_hbm, ids_hbm, out_hbm,
             acc_ref, xbuf0, xbuf1, idb0, idb1, bounds_vmem, counts_smem,
             sem0, sem1):
    w = lax.axis_index("c") * 16 + lax.axis_index("s")
    s_lo = w * SEG_PER

    pltpu.sync_copy(bounds_hbm, bounds_vmem.at[pl.ds(0, NB)])
    bv = bounds_vmem[pl.ds(w, L)]
    r_lo = bv[0]
    r_hi = bv[1]

    @pl.loop(0, SEG_PER * D, step=L)
    def _(o):
        acc_ref[pl.ds(o, L)] = jnp.zeros((L,), jnp.float32)

    @pl.loop(0, SEG_PER)
    def _(k):
        counts_smem[k] = 0

    base = (r_lo // 16) * 16
    nchunks = (r_hi - base + C - 1) // C

    def chunk_start(cix):
        return jnp.minimum(base + cix * C, N - C)

    def issue(cix, xb, ib, sem):
        start = chunk_start(cix)
        pltpu.async_copy(x_hbm.at[pl.ds(start * D, C * D)], xb, sem)
        pltpu.async_copy(ids_hbm.at[pl.ds(start, C)], ib.at[pl.ds(0, C)], sem)

    def wait(xb, ib, sem):
        start = chunk_start(0)
        pltpu.make_async_copy(x_hbm.at[pl.ds(start * D, C * D)], xb, sem).wait()
        pltpu.make_async_copy(
            ids_hbm.at[pl.ds(start, C)], ib.at[pl.ds(0, C)], sem).wait()

    def process(cix, xb, ib):
        start = chunk_start(cix)
        lo = jnp.maximum(r_lo, base + cix * C)
        hi = jnp.minimum(base + (cix + 1) * C, r_hi)

        def row_body(r, _):
            i = r - start
            lid = ib[pl.ds(i, L)][0] - s_lo
            counts_smem[lid] = counts_smem[lid] + 1
            off = lid * D
            ioff = i * D
            for j in range(D // L):
                v = xb[pl.ds(ioff + j * L, L)]
                plsc.addupdate(acc_ref.at[pl.ds(off + j * L, L)], v)
            return 0

        lax.fori_loop(lo, hi, row_body, 0)

    @pl.when(nchunks > 0)
    def _():
        issue(0, xbuf0, idb0, sem0)

    def pair_body(p, _):
        c0 = 2 * p
        c1 = c0 + 1
        wait(xbuf0, idb0, sem0)

        @pl.when(c1 < nchunks)
        def _():
            issue(c1, xbuf1, idb1, sem1)

        process(c0, xbuf0, idb0)

        @pl.when(c1 < nchunks)
        def _():
            wait(xbuf1, idb1, sem1)

            @pl.when(c1 + 1 < nchunks)
            def _():
                issue(c1 + 1, xbuf0, idb0, sem0)

            process(c1, xbuf1, idb1)

        return 0

    lax.fori_loop(0, (nchunks + 1) // 2, pair_body, 0)

    @pl.loop(0, SEG_PER)
    def _(k):
        cntf = jnp.maximum(counts_smem[k], 1).astype(jnp.float32)
        off = k * D
        for j in range(D // L):
            v = acc_ref[pl.ds(off + j * L, L)]
            acc_ref[pl.ds(off + j * L, L)] = v / cntf

    pltpu.sync_copy(acc_ref, out_hbm.at[pl.ds(s_lo * D, SEG_PER * D)])


def _make_sc_kernel(interpret=False):
    mesh = plsc.VectorSubcoreMesh(
        core_axis_name="c", subcore_axis_name="s", num_cores=2, num_subcores=16
    )
    return pl.kernel(
        _sc_body,
        out_type=jax.ShapeDtypeStruct((S_PAD * D,), jnp.float32),
        mesh=mesh,
        scratch_types=[
            pltpu.VMEM((SEG_PER * D,), jnp.float32),
            pltpu.VMEM((C * D,), jnp.float32),
            pltpu.VMEM((C * D,), jnp.float32),
            pltpu.VMEM((C + L,), jnp.int32),
            pltpu.VMEM((C + L,), jnp.int32),
            pltpu.VMEM((NB + L,), jnp.int32),
            pltpu.SMEM((SEG_PER,), jnp.int32),
            pltpu.SemaphoreType.DMA,
            pltpu.SemaphoreType.DMA,
        ],
        interpret=interpret,
    )


@jax.jit
def kernel(X, cluster_assignment):
    ids = cluster_assignment.astype(jnp.int32)
    queries = jnp.arange(NB, dtype=jnp.int32) * SEG_PER
    bounds = jnp.searchsorted(ids, queries, side="left").astype(jnp.int32)
    out = _make_sc_kernel()(X.reshape(N * D), bounds, ids)
    return out.reshape(S_PAD, D)[:S]


# speedup vs baseline: 7.4290x; 2.5743x over previous
import jax
import jax.numpy as jnp
from jax import lax
from jax.experimental import pallas as pl
from jax.experimental.pallas import tpu as pltpu
from jax.experimental.pallas import tpu_sc as plsc

N = 320000
D = 128
S = 10000
L = 16
NW = 32
SEG_PER = 313
S_PAD = NW * SEG_PER
C = 320
NB = 48


def _sc_body(x_hbm, bounds_hbm, ids_hbm, out_hbm,
             acc_ref, xbuf0, xbuf1, idb0, idb1, bounds_vmem, counts_smem,
             sem0, sem1):
    w = lax.axis_index("c") * 16 + lax.axis_index("s")
    s_lo = w * SEG_PER

    pltpu.sync_copy(bounds_hbm, bounds_vmem.at[pl.ds(0, NB)])
    bv = bounds_vmem[pl.ds(w, L)]
    r_lo = bv[0]
    r_hi = bv[1]

    @pl.loop(0, SEG_PER * D, step=L)
    def _(o):
        acc_ref[pl.ds(o, L)] = jnp.zeros((L,), jnp.float32)

    @pl.loop(0, SEG_PER)
    def _(k):
        counts_smem[k] = 0

    base = (r_lo // 16) * 16
    nchunks = (r_hi - base + C - 1) // C

    def chunk_start(cix):
        return jnp.minimum(base + cix * C, N - C)

    def issue(cix, xb, ib, sem):
        start = chunk_start(cix)
        pltpu.async_copy(x_hbm.at[pl.ds(start * D, C * D)], xb, sem)
        pltpu.async_copy(ids_hbm.at[pl.ds(start, C)], ib.at[pl.ds(0, C)], sem)

    def wait(xb, ib, sem):
        start = chunk_start(0)
        pltpu.make_async_copy(x_hbm.at[pl.ds(start * D, C * D)], xb, sem).wait()
        pltpu.make_async_copy(
            ids_hbm.at[pl.ds(start, C)], ib.at[pl.ds(0, C)], sem).wait()

    def process(cix, xb, ib):
        start = chunk_start(cix)
        lo = jnp.maximum(r_lo, base + cix * C)
        hi = jnp.minimum(base + (cix + 1) * C, r_hi)

        def accum_row(ioff, off, lid):
            counts_smem[lid] = counts_smem[lid] + 1
            vs = [xb[pl.ds(ioff + j * L, L)] for j in range(D // L)]
            for j, v in enumerate(vs):
                plsc.addupdate(acc_ref.at[pl.ds(off + j * L, L)], v)

        def row_body(r, _):
            i = r - start
            lid = ib[pl.ds(i, L)][0] - s_lo
            accum_row(i * D, lid * D, lid)
            return 0

        lo16 = ((lo + L - 1) // L) * L
        hi16 = (hi // L) * L
        lax.fori_loop(lo, jnp.minimum(lo16, hi), row_body, 0)

        def group_body(g, _):
            i0 = g * L - start
            lidv = ib[pl.ds(i0, L)] - s_lo
            for k in range(L):
                lid = lidv[k]
                accum_row((i0 + k) * D, lid * D, lid)
            return 0

        lax.fori_loop(lo16 // L, hi16 // L, group_body, 0)

        lax.fori_loop(jnp.maximum(lo16, hi16), hi, row_body, 0)

    @pl.when(nchunks > 0)
    def _():
        issue(0, xbuf0, idb0, sem0)

    def pair_body(p, _):
        c0 = 2 * p
        c1 = c0 + 1
        wait(xbuf0, idb0, sem0)

        @pl.when(c1 < nchunks)
        def _():
            issue(c1, xbuf1, idb1, sem1)

        process(c0, xbuf0, idb0)

        @pl.when(c1 < nchunks)
        def _():
            wait(xbuf1, idb1, sem1)

            @pl.when(c1 + 1 < nchunks)
            def _():
                issue(c1 + 1, xbuf0, idb0, sem0)

            process(c1, xbuf1, idb1)

        return 0

    lax.fori_loop(0, (nchunks + 1) // 2, pair_body, 0)

    @pl.loop(0, SEG_PER)
    def _(k):
        cntf = jnp.maximum(counts_smem[k], 1).astype(jnp.float32)
        off = k * D
        for j in range(D // L):
            v = acc_ref[pl.ds(off + j * L, L)]
            acc_ref[pl.ds(off + j * L, L)] = v / cntf

    pltpu.sync_copy(acc_ref, out_hbm.at[pl.ds(s_lo * D, SEG_PER * D)])


def _make_sc_kernel(interpret=False):
    mesh = plsc.VectorSubcoreMesh(
        core_axis_name="c", subcore_axis_name="s", num_cores=2, num_subcores=16
    )
    return pl.kernel(
        _sc_body,
        out_type=jax.ShapeDtypeStruct((S_PAD * D,), jnp.float32),
        mesh=mesh,
        scratch_types=[
            pltpu.VMEM((SEG_PER * D,), jnp.float32),
            pltpu.VMEM((C * D,), jnp.float32),
            pltpu.VMEM((C * D,), jnp.float32),
            pltpu.VMEM((C + L,), jnp.int32),
            pltpu.VMEM((C + L,), jnp.int32),
            pltpu.VMEM((NB + L,), jnp.int32),
            pltpu.SMEM((SEG_PER,), jnp.int32),
            pltpu.SemaphoreType.DMA,
            pltpu.SemaphoreType.DMA,
        ],
        interpret=interpret,
    )


@jax.jit
def kernel(X, cluster_assignment):
    ids = cluster_assignment.astype(jnp.int32)
    queries = jnp.arange(NB, dtype=jnp.int32) * SEG_PER
    bounds = jnp.searchsorted(ids, queries, side="left").astype(jnp.int32)
    out = _make_sc_kernel()(X.reshape(N * D), bounds, ids)
    return out.reshape(S_PAD, D)[:S]
